# baseline (speedup 1.0000x reference)
import functools

import jax
import jax.numpy as jnp
from jax import lax
from jax.experimental import pallas as pl
from jax.experimental.pallas import tpu as pltpu
from jax.experimental.pallas import tpu_sc as plsc

N = 10000
E = 320000
F_IN = 128
HID = 64
NCLS = 10

NP = 10112
NSC, NT = 2, 16
CH = 128
CPT = 80
EPT = CH * CPT
EPAD = NSC * NT * EPT
ZR = NP // NT
NSLOT = 8
NAHEAD = 4

RB = 1024


@functools.lru_cache(maxsize=None)
def _s_pass(D):
    G = 2 if D > 32 else 1
    CD = D // G
    mesh = plsc.VectorSubcoreMesh(core_axis_name="c", subcore_axis_name="s")

    @functools.partial(
        pl.kernel,
        out_type=jax.ShapeDtypeStruct((NSC, G, NP, CD), jnp.float32),
        mesh=mesh,
        scratch_types=[
            pltpu.VMEM((CPT, CH), jnp.int32),
            pltpu.VMEM((CPT, CH), jnp.int32),
            pltpu.VMEM((NSLOT, CH, CD), jnp.float32),
            pltpu.VMEM_SHARED((NP, CD), jnp.float32),
            pltpu.VMEM_SHARED((NP, CD), jnp.float32),
        ] + [pltpu.SemaphoreType.DMA] * (2 * NSLOT),
        compiler_params=pltpu.CompilerParams(use_tc_tiling_on_sc=False, skip_device_barrier=True),
    )
    def body(table, rowp3, colp3, zeros, out, ridx, cidx, rows, tbl, acc,
             *sems):
        gsems, ssems = sems[:NSLOT], sems[NSLOT:]
        c = lax.axis_index("c")
        s = lax.axis_index("s")
        wid = c * NT + s
        pltpu.sync_copy(rowp3.at[wid], ridx)
        pltpu.sync_copy(colp3.at[wid], cidx)

        def group(g, gcarry):
            pltpu.sync_copy(table.at[g, pl.ds(s * ZR, ZR)],
                            tbl.at[pl.ds(s * ZR, ZR)])
            pltpu.sync_copy(zeros.at[g, pl.ds(s * ZR, ZR)],
                            acc.at[pl.ds(s * ZR, ZR)])
            plsc.subcore_barrier()
            for b in range(NAHEAD):
                pltpu.async_copy(tbl.at[ridx.at[b]], rows.at[b], gsems[b])

            def step(j, carry):
                for b in range(NSLOT):
                    i = j * NSLOT + b
                    bp = (b + NAHEAD) % NSLOT
                    pltpu.make_async_copy(tbl.at[ridx.at[i]], rows.at[b],
                                          gsems[b]).wait()
                    pltpu.async_copy(rows.at[b], acc.at[cidx.at[i]],
                                     ssems[b], add=True)

                    @pl.when(i + NAHEAD < CPT)
                    def _():
                        @pl.when(i >= NAHEAD)
                        def _():
                            pltpu.make_async_copy(
                                rows.at[bp], acc.at[cidx.at[0]],
                                ssems[bp]).wait()

                        pltpu.async_copy(tbl.at[ridx.at[i + NAHEAD]],
                                         rows.at[bp], gsems[bp])
                return carry

            lax.fori_loop(0, CPT // NSLOT, step, 0)
            for b in range(NSLOT):
                pltpu.make_async_copy(rows.at[b], acc.at[cidx.at[0]],
                                      ssems[b]).wait()
            plsc.subcore_barrier()
            pltpu.sync_copy(acc.at[pl.ds(s * ZR, ZR)],
                            out.at[c, g, pl.ds(s * ZR, ZR)])
            return gcarry

        lax.fori_loop(0, G, group, 0)

    return body


def _l2_pass():
    mesh = plsc.VectorSubcoreMesh(core_axis_name="c", subcore_axis_name="s")

    @functools.partial(
        pl.kernel,
        out_type=jax.ShapeDtypeStruct((NSC, NP, 16), jnp.float32),
        mesh=mesh,
        scratch_types=[
            pltpu.VMEM((2 * CPT, CH), jnp.int32),
            pltpu.VMEM((2 * CPT, CH), jnp.int32),
            pltpu.VMEM((NSLOT, CH, 16), jnp.float32),
            pltpu.VMEM((ZR, 16), jnp.float32),
            pltpu.VMEM((ZR, 16), jnp.float32),
            pltpu.VMEM((ZR, 16), jnp.float32),
            pltpu.VMEM_SHARED((NP, 16), jnp.float32),
            pltpu.VMEM_SHARED((NP, 16), jnp.float32),
        ] + [pltpu.SemaphoreType.DMA] * (2 * NSLOT),
        compiler_params=pltpu.CompilerParams(use_tc_tiling_on_sc=False,
                                             skip_device_barrier=True),
    )
    def body(y2b, v1d, d2b2, rowp3, colp3, zeros, out, ridx, cidx, rows,
             ew_g, ew_v, ew_d, tbl, acc, *sems):
        gsems, ssems = sems[:NSLOT], sems[NSLOT:]
        c = lax.axis_index("c")
        s = lax.axis_index("s")
        pltpu.sync_copy(rowp3.at[2 * s], ridx.at[pl.ds(0, CPT)])
        pltpu.sync_copy(rowp3.at[2 * s + 1], ridx.at[pl.ds(CPT, CPT)])
        pltpu.sync_copy(colp3.at[2 * s], cidx.at[pl.ds(0, CPT)])
        pltpu.sync_copy(colp3.at[2 * s + 1], cidx.at[pl.ds(CPT, CPT)])
        pltpu.sync_copy(y2b.at[pl.ds(s * ZR, ZR)], tbl.at[pl.ds(s * ZR, ZR)])
        pltpu.sync_copy(zeros.at[pl.ds(s * ZR, ZR)], acc.at[pl.ds(s * ZR, ZR)])
        plsc.subcore_barrier()

        def run_ring(base, nchunks):
            for b in range(NAHEAD):
                pltpu.async_copy(tbl.at[ridx.at[base + b]], rows.at[b],
                                 gsems[b])

            def step(j, carry):
                for b in range(NSLOT):
                    i = j * NSLOT + b
                    bp = (b + NAHEAD) % NSLOT
                    pltpu.make_async_copy(tbl.at[ridx.at[base + i]],
                                          rows.at[b], gsems[b]).wait()
                    pltpu.async_copy(rows.at[b], acc.at[cidx.at[base + i]],
                                     ssems[b], add=True)

                    @pl.when(i + NAHEAD < nchunks)
                    def _():
                        @pl.when(i >= NAHEAD)
                        def _():
                            pltpu.make_async_copy(
                                rows.at[bp], acc.at[cidx.at[base]],
                                ssems[bp]).wait()

                        pltpu.async_copy(tbl.at[ridx.at[base + i + NAHEAD]],
                                         rows.at[bp], gsems[bp])
                return carry

            lax.fori_loop(0, nchunks // NSLOT, step, 0)
            for b in range(NSLOT):
                pltpu.make_async_copy(rows.at[b], acc.at[cidx.at[base]],
                                      ssems[b]).wait()

        run_ring(0, 2 * CPT)
        plsc.subcore_barrier()

        pltpu.sync_copy(acc.at[pl.ds(s * ZR, ZR)], ew_g)
        pltpu.sync_copy(v1d.at[pl.ds(s * ZR, ZR)], ew_v)
        pltpu.sync_copy(d2b2.at[pl.ds(s * ZR, ZR)], ew_d)

        def ew(r, carry):
            ew_g[r, :] = ew_v[r, :] - ew_d[r, :] * ew_g[r, :]
            return carry

        lax.fori_loop(0, ZR, ew, 0)
        pltpu.sync_copy(ew_g, tbl.at[pl.ds(s * ZR, ZR)])
        pltpu.sync_copy(zeros.at[pl.ds(s * ZR, ZR)], acc.at[pl.ds(s * ZR, ZR)])
        plsc.subcore_barrier()

        run_ring(c * CPT, CPT)
        plsc.subcore_barrier()
        pltpu.sync_copy(acc.at[pl.ds(s * ZR, ZR)], out.at[c, pl.ds(s * ZR, ZR)])

    return body


def _deg_pass():
    mesh = plsc.VectorSubcoreMesh(core_axis_name="c", subcore_axis_name="s")

    @functools.partial(
        pl.kernel,
        out_type=jax.ShapeDtypeStruct((NSC, NP, 16), jnp.float32),
        mesh=mesh,
        scratch_types=[
            pltpu.VMEM((CPT, CH), jnp.int32),
            pltpu.VMEM((CH, 16), jnp.float32),
            pltpu.VMEM_SHARED((NP, 16), jnp.float32),
            pltpu.SemaphoreType.DMA,
        ],
        compiler_params=pltpu.CompilerParams(use_tc_tiling_on_sc=False, skip_device_barrier=True),
    )
    def body(rowp3, onesc, zeros, out, ridx, ones_v, acc, ssem):
        c = lax.axis_index("c")
        s = lax.axis_index("s")
        wid = c * NT + s
        pltpu.sync_copy(rowp3.at[wid], ridx)
        pltpu.sync_copy(onesc, ones_v)
        pltpu.sync_copy(zeros.at[pl.ds(s * ZR, ZR)], acc.at[pl.ds(s * ZR, ZR)])
        plsc.subcore_barrier()

        def step(i, carry):
            pltpu.async_copy(ones_v, acc.at[ridx.at[i]], ssem, add=True).wait()
            return carry

        lax.fori_loop(0, CPT, step, 0)
        plsc.subcore_barrier()
        pltpu.sync_copy(acc.at[pl.ds(s * ZR, ZR)], out.at[c, pl.ds(s * ZR, ZR)])

    return body


def _dinv_of(degp):
    deg = degp[0] + degp[1]
    dinv = jnp.where(deg > 0, lax.rsqrt(jnp.maximum(deg, 1e-12)), 0.0)
    return dinv[:, 0:1]


def _mm1_body(x_ref, w_ref, o_ref):
    o_ref[...] = jnp.dot(x_ref[...], w_ref[...],
                         preferred_element_type=jnp.float32)


def _mm1(x, wc1):
    return pl.pallas_call(
        _mm1_body,
        grid=(pl.cdiv(NP, RB),),
        in_specs=[
            pl.BlockSpec((RB, F_IN), lambda i: (i, 0)),
            pl.BlockSpec((F_IN, 3 * HID), lambda i: (0, 0)),
        ],
        out_specs=pl.BlockSpec((RB, 3 * HID), lambda i: (i, 0)),
        out_shape=jax.ShapeDtypeStruct((NP, 3 * HID), jnp.float32),
    )(x, wc1)


def _prep_body(xw_ref, degp_ref, y2_ref, u1d_ref):
    dinv = _dinv_of(degp_ref[...])
    xw = xw_ref[...]
    u1d_ref[...] = jnp.stack(
        [dinv * xw[:, HID:HID + 32], dinv * xw[:, HID + 32:2 * HID]], axis=0)
    y2_ref[...] = jnp.stack(
        [dinv * xw[:, 2 * HID:2 * HID + 32], dinv * xw[:, 2 * HID + 32:]],
        axis=0)


def _prep(xw, degp):
    return pl.pallas_call(
        _prep_body,
        grid=(pl.cdiv(NP, RB),),
        in_specs=[
            pl.BlockSpec((RB, 3 * HID), lambda i: (i, 0)),
            pl.BlockSpec((NSC, RB, 16), lambda i: (0, i, 0)),
        ],
        out_specs=[
            pl.BlockSpec((2, RB, 32), lambda i: (0, i, 0)),
            pl.BlockSpec((2, RB, 32), lambda i: (0, i, 0)),
        ],
        out_shape=[
            jax.ShapeDtypeStruct((2, NP, 32), jnp.float32),
            jax.ShapeDtypeStruct((2, NP, 32), jnp.float32),
        ],
    )(xw, degp)


def _comb_body(gp_ref, u_ref, degp_ref, o_ref):
    dinv = _dinv_of(degp_ref[...])
    d2 = (dinv * dinv)[None]
    o_ref[...] = u_ref[...] - 2.0 * d2 * (gp_ref[0] + gp_ref[1])


def _comb(gp, u, degp):
    G, _, CD = u.shape
    return pl.pallas_call(
        _comb_body,
        grid=(pl.cdiv(NP, RB),),
        in_specs=[
            pl.BlockSpec((NSC, G, RB, CD), lambda i: (0, 0, i, 0)),
            pl.BlockSpec((G, RB, CD), lambda i: (0, i, 0)),
            pl.BlockSpec((NSC, RB, 16), lambda i: (0, i, 0)),
        ],
        out_specs=pl.BlockSpec((G, RB, CD), lambda i: (0, i, 0)),
        out_shape=jax.ShapeDtypeStruct((G, NP, CD), jnp.float32),
    )(gp, u, degp)


def _layer2_body(xw_ref, g1p_ref, degp_ref, b1_ref, wc2_ref, b2_ref,
                 b2v_ref, v1d_ref, y2b_ref, d2b2_ref):
    dinv = _dinv_of(degp_ref[...])
    a = xw_ref[:, 0:HID]
    g1 = g1p_ref[0] + g1p_ref[1]
    g1cat = jnp.concatenate([g1[0], g1[1]], axis=1)
    h = jnp.maximum(a + b1_ref[...] - dinv * g1cat, 0.0)
    hw = jnp.dot(h, wc2_ref[...], preferred_element_type=jnp.float32)
    b2v_ref[...] = hw[:, 0:16] + b2_ref[...]
    v1d_ref[...] = dinv * hw[:, 16:32]
    y2b_ref[...] = dinv * hw[:, 32:48]
    d2b2_ref[...] = jnp.broadcast_to(2.0 * dinv * dinv, d2b2_ref.shape)


def _layer2(xw, g1p, degp, b1r, wc2, b2r):
    return pl.pallas_call(
        _layer2_body,
        grid=(pl.cdiv(NP, RB),),
        in_specs=[
            pl.BlockSpec((RB, 3 * HID), lambda i: (i, 0)),
            pl.BlockSpec((NSC, 2, RB, 32), lambda i: (0, 0, i, 0)),
            pl.BlockSpec((NSC, RB, 16), lambda i: (0, i, 0)),
            pl.BlockSpec((1, HID), lambda i: (0, 0)),
            pl.BlockSpec((HID, 48), lambda i: (0, 0)),
            pl.BlockSpec((1, 16), lambda i: (0, 0)),
        ],
        out_specs=[
            pl.BlockSpec((RB, 16), lambda i: (i, 0)),
            pl.BlockSpec((RB, 16), lambda i: (i, 0)),
            pl.BlockSpec((RB, 16), lambda i: (i, 0)),
            pl.BlockSpec((RB, 16), lambda i: (i, 0)),
        ],
        out_shape=[
            jax.ShapeDtypeStruct((NP, 16), jnp.float32),
            jax.ShapeDtypeStruct((NP, 16), jnp.float32),
            jax.ShapeDtypeStruct((NP, 16), jnp.float32),
            jax.ShapeDtypeStruct((NP, 16), jnp.float32),
        ],
    )(xw, g1p, degp, b1r, wc2, b2r)


def _final_body(b2v_ref, g1bp_ref, degp_ref, o_ref):
    dinv = _dinv_of(degp_ref[...])
    o_ref[...] = b2v_ref[...] - dinv * (g1bp_ref[0] + g1bp_ref[1])


def _final(b2v, g1bp, degp):
    return pl.pallas_call(
        _final_body,
        grid=(pl.cdiv(NP, RB),),
        in_specs=[
            pl.BlockSpec((RB, 16), lambda i: (i, 0)),
            pl.BlockSpec((NSC, RB, 16), lambda i: (0, i, 0)),
            pl.BlockSpec((NSC, RB, 16), lambda i: (0, i, 0)),
        ],
        out_specs=pl.BlockSpec((RB, 16), lambda i: (i, 0)),
        out_shape=jax.ShapeDtypeStruct((NP, 16), jnp.float32),
    )(b2v, g1bp, degp)


def kernel(x, edge_index, W1, b1, W2, b2):
    row, col = edge_index[0], edge_index[1]
    pad = EPAD - E
    trash = jnp.full((pad,), N, dtype=jnp.int32)
    rowp = jnp.concatenate([row, trash]).reshape(NSC * NT, CPT, CH)
    colp = jnp.concatenate([col, trash]).reshape(NSC * NT, CPT, CH)

    wc1 = jnp.concatenate([W1[0] - W1[2], W1[1], W1[2]], axis=1)
    wc2 = jnp.zeros((HID, 48), jnp.float32)
    wc2 = wc2.at[:, 0:NCLS].set(W2[0] - W2[2])
    wc2 = wc2.at[:, 16:16 + NCLS].set(W2[1])
    wc2 = wc2.at[:, 32:32 + NCLS].set(W2[2])
    b1r = b1.reshape(1, HID)
    b2r = jnp.zeros((1, 16), jnp.float32).at[0, 0:NCLS].set(b2)

    onesc = jnp.ones((CH, 16), jnp.float32)
    zdeg = jnp.zeros((NP, 16), jnp.float32)
    zeros64 = jnp.zeros((2, NP, 32), jnp.float32)

    degp = _deg_pass()(rowp, onesc, zdeg)
    xw = _mm1(x, wc1)

    y2, u1d = _prep(xw, degp)
    g2p = _s_pass(HID)(y2, rowp, colp, zeros64)
    inner = _comb(g2p, u1d, degp)
    g1p = _s_pass(HID)(inner, rowp, colp, zeros64)

    b2v, v1d, y2b, d2b2 = _layer2(xw, g1p, degp, b1r, wc2, b2r)
    g1bp = _l2_pass()(y2b, v1d, d2b2, rowp, colp, zdeg)

    out16 = _final(b2v, g1bp, degp)
    return (out16[:N, :NCLS], edge_index)


# speedup vs baseline: 1.0077x; 1.0077x over previous
import functools

import jax
import jax.numpy as jnp
from jax import lax
from jax.experimental import pallas as pl
from jax.experimental.pallas import tpu as pltpu
from jax.experimental.pallas import tpu_sc as plsc

N = 10000
E = 320000
F_IN = 128
HID = 64
NCLS = 10

NP = 10112
NSC, NT = 2, 16
CH = 128
CPT = 80
EPT = CH * CPT
EPAD = NSC * NT * EPT
ZR = NP // NT
NSLOT = 8
NAHEAD = 4

RB = 1024


@functools.lru_cache(maxsize=None)
def _s_pass(D):
    G = 2 if D > 32 else 1
    CD = D // G
    mesh = plsc.VectorSubcoreMesh(core_axis_name="c", subcore_axis_name="s")

    @functools.partial(
        pl.kernel,
        out_type=jax.ShapeDtypeStruct((NSC, G, NP, CD), jnp.float32),
        mesh=mesh,
        scratch_types=[
            pltpu.VMEM((CPT, CH), jnp.int32),
            pltpu.VMEM((CPT, CH), jnp.int32),
            pltpu.VMEM((NSLOT, CH, CD), jnp.float32),
            pltpu.VMEM_SHARED((NP, CD), jnp.float32),
            pltpu.VMEM_SHARED((NP, CD), jnp.float32),
        ] + [pltpu.SemaphoreType.DMA] * (2 * NSLOT),
        compiler_params=pltpu.CompilerParams(use_tc_tiling_on_sc=False, skip_device_barrier=True),
    )
    def body(table, rowp3, colp3, zeros, out, ridx, cidx, rows, tbl, acc,
             *sems):
        gsems, ssems = sems[:NSLOT], sems[NSLOT:]
        c = lax.axis_index("c")
        s = lax.axis_index("s")
        wid = c * NT + s
        pltpu.sync_copy(rowp3.at[wid], ridx)
        pltpu.sync_copy(colp3.at[wid], cidx)

        def group(g, gcarry):
            pltpu.sync_copy(table.at[g, pl.ds(s * ZR, ZR)],
                            tbl.at[pl.ds(s * ZR, ZR)])
            pltpu.sync_copy(zeros.at[g, pl.ds(s * ZR, ZR)],
                            acc.at[pl.ds(s * ZR, ZR)])
            plsc.subcore_barrier()
            for b in range(NAHEAD):
                pltpu.async_copy(tbl.at[ridx.at[b]], rows.at[b], gsems[b])

            def step(j, carry):
                for b in range(NSLOT):
                    i = j * NSLOT + b
                    bp = (b + NAHEAD) % NSLOT
                    pltpu.make_async_copy(tbl.at[ridx.at[i]], rows.at[b],
                                          gsems[b]).wait()
                    pltpu.async_copy(rows.at[b], acc.at[cidx.at[i]],
                                     ssems[b], add=True)

                    @pl.when(i + NAHEAD < CPT)
                    def _():
                        @pl.when(i >= NAHEAD)
                        def _():
                            pltpu.make_async_copy(
                                rows.at[bp], acc.at[cidx.at[0]],
                                ssems[bp]).wait()

                        pltpu.async_copy(tbl.at[ridx.at[i + NAHEAD]],
                                         rows.at[bp], gsems[bp])
                return carry

            lax.fori_loop(0, CPT // NSLOT, step, 0)
            for b in range(NSLOT):
                pltpu.make_async_copy(rows.at[b], acc.at[cidx.at[0]],
                                      ssems[b]).wait()
            plsc.subcore_barrier()
            pltpu.sync_copy(acc.at[pl.ds(s * ZR, ZR)],
                            out.at[c, g, pl.ds(s * ZR, ZR)])
            return gcarry

        lax.fori_loop(0, G, group, 0)

    return body


def _l2_pass():
    mesh = plsc.VectorSubcoreMesh(core_axis_name="c", subcore_axis_name="s")

    @functools.partial(
        pl.kernel,
        out_type=jax.ShapeDtypeStruct((NSC, NP, 16), jnp.float32),
        mesh=mesh,
        scratch_types=[
            pltpu.VMEM((2 * CPT, CH), jnp.int32),
            pltpu.VMEM((2 * CPT, CH), jnp.int32),
            pltpu.VMEM((NSLOT, CH, 16), jnp.float32),
            pltpu.VMEM((ZR, 16), jnp.float32),
            pltpu.VMEM((ZR, 16), jnp.float32),
            pltpu.VMEM((ZR, 16), jnp.float32),
            pltpu.VMEM_SHARED((NP, 16), jnp.float32),
            pltpu.VMEM_SHARED((NP, 16), jnp.float32),
        ] + [pltpu.SemaphoreType.DMA] * (2 * NSLOT),
        compiler_params=pltpu.CompilerParams(use_tc_tiling_on_sc=False,
                                             skip_device_barrier=True),
    )
    def body(y2b, v1d, d2b2, rowp3, colp3, zeros, out, ridx, cidx, rows,
             ew_g, ew_v, ew_d, tbl, acc, *sems):
        gsems, ssems = sems[:NSLOT], sems[NSLOT:]
        c = lax.axis_index("c")
        s = lax.axis_index("s")
        pltpu.sync_copy(rowp3.at[2 * s], ridx.at[pl.ds(0, CPT)])
        pltpu.sync_copy(rowp3.at[2 * s + 1], ridx.at[pl.ds(CPT, CPT)])
        pltpu.sync_copy(colp3.at[2 * s], cidx.at[pl.ds(0, CPT)])
        pltpu.sync_copy(colp3.at[2 * s + 1], cidx.at[pl.ds(CPT, CPT)])
        pltpu.sync_copy(y2b.at[pl.ds(s * ZR, ZR)], tbl.at[pl.ds(s * ZR, ZR)])
        pltpu.sync_copy(zeros.at[pl.ds(s * ZR, ZR)], acc.at[pl.ds(s * ZR, ZR)])
        plsc.subcore_barrier()

        def run_ring(base, nchunks):
            for b in range(NAHEAD):
                pltpu.async_copy(tbl.at[ridx.at[base + b]], rows.at[b],
                                 gsems[b])

            def step(j, carry):
                for b in range(NSLOT):
                    i = j * NSLOT + b
                    bp = (b + NAHEAD) % NSLOT
                    pltpu.make_async_copy(tbl.at[ridx.at[base + i]],
                                          rows.at[b], gsems[b]).wait()
                    pltpu.async_copy(rows.at[b], acc.at[cidx.at[base + i]],
                                     ssems[b], add=True)

                    @pl.when(i + NAHEAD < nchunks)
                    def _():
                        @pl.when(i >= NAHEAD)
                        def _():
                            pltpu.make_async_copy(
                                rows.at[bp], acc.at[cidx.at[base]],
                                ssems[bp]).wait()

                        pltpu.async_copy(tbl.at[ridx.at[base + i + NAHEAD]],
                                         rows.at[bp], gsems[bp])
                return carry

            lax.fori_loop(0, nchunks // NSLOT, step, 0)
            for b in range(NSLOT):
                pltpu.make_async_copy(rows.at[b], acc.at[cidx.at[base]],
                                      ssems[b]).wait()

        run_ring(0, 2 * CPT)
        plsc.subcore_barrier()

        pltpu.sync_copy(acc.at[pl.ds(s * ZR, ZR)], ew_g)
        pltpu.sync_copy(v1d.at[pl.ds(s * ZR, ZR)], ew_v)
        pltpu.sync_copy(d2b2.at[pl.ds(s * ZR, ZR)], ew_d)

        def ew(r4, carry):
            for k in range(4):
                r = r4 * 4 + k
                ew_g[r, :] = ew_v[r, :] - ew_d[r, :] * ew_g[r, :]
            return carry

        lax.fori_loop(0, ZR // 4, ew, 0)
        pltpu.sync_copy(ew_g, tbl.at[pl.ds(s * ZR, ZR)])
        pltpu.sync_copy(zeros.at[pl.ds(s * ZR, ZR)], acc.at[pl.ds(s * ZR, ZR)])
        plsc.subcore_barrier()

        run_ring(c * CPT, CPT)
        plsc.subcore_barrier()
        pltpu.sync_copy(acc.at[pl.ds(s * ZR, ZR)], out.at[c, pl.ds(s * ZR, ZR)])

    return body


def _deg_pass():
    mesh = plsc.VectorSubcoreMesh(core_axis_name="c", subcore_axis_name="s")

    @functools.partial(
        pl.kernel,
        out_type=jax.ShapeDtypeStruct((NSC, NP, 16), jnp.float32),
        mesh=mesh,
        scratch_types=[
            pltpu.VMEM((CPT, CH), jnp.int32),
            pltpu.VMEM((CH, 16), jnp.float32),
            pltpu.VMEM_SHARED((NP, 16), jnp.float32),
            pltpu.SemaphoreType.DMA,
        ],
        compiler_params=pltpu.CompilerParams(use_tc_tiling_on_sc=False, skip_device_barrier=True),
    )
    def body(rowp3, onesc, zeros, out, ridx, ones_v, acc, ssem):
        c = lax.axis_index("c")
        s = lax.axis_index("s")
        wid = c * NT + s
        pltpu.sync_copy(rowp3.at[wid], ridx)
        pltpu.sync_copy(onesc, ones_v)
        pltpu.sync_copy(zeros.at[pl.ds(s * ZR, ZR)], acc.at[pl.ds(s * ZR, ZR)])
        plsc.subcore_barrier()

        def step(i, carry):
            pltpu.async_copy(ones_v, acc.at[ridx.at[i]], ssem, add=True).wait()
            return carry

        lax.fori_loop(0, CPT, step, 0)
        plsc.subcore_barrier()
        pltpu.sync_copy(acc.at[pl.ds(s * ZR, ZR)], out.at[c, pl.ds(s * ZR, ZR)])

    return body


def _dinv_of(degp):
    deg = degp[0] + degp[1]
    dinv = jnp.where(deg > 0, lax.rsqrt(jnp.maximum(deg, 1e-12)), 0.0)
    return dinv[:, 0:1]


def _mm1_body(x_ref, w_ref, o_ref):
    o_ref[...] = jnp.dot(x_ref[...], w_ref[...],
                         preferred_element_type=jnp.float32)


def _mm1(x, wc1):
    return pl.pallas_call(
        _mm1_body,
        grid=(pl.cdiv(NP, RB),),
        in_specs=[
            pl.BlockSpec((RB, F_IN), lambda i: (i, 0)),
            pl.BlockSpec((F_IN, 3 * HID), lambda i: (0, 0)),
        ],
        out_specs=pl.BlockSpec((RB, 3 * HID), lambda i: (i, 0)),
        out_shape=jax.ShapeDtypeStruct((NP, 3 * HID), jnp.float32),
    )(x, wc1)


def _prep_body(xw_ref, degp_ref, y2_ref, u1d_ref):
    dinv = _dinv_of(degp_ref[...])
    xw = xw_ref[...]
    u1d_ref[...] = jnp.stack(
        [dinv * xw[:, HID:HID + 32], dinv * xw[:, HID + 32:2 * HID]], axis=0)
    y2_ref[...] = jnp.stack(
        [dinv * xw[:, 2 * HID:2 * HID + 32], dinv * xw[:, 2 * HID + 32:]],
        axis=0)


def _prep(xw, degp):
    return pl.pallas_call(
        _prep_body,
        grid=(pl.cdiv(NP, RB),),
        in_specs=[
            pl.BlockSpec((RB, 3 * HID), lambda i: (i, 0)),
            pl.BlockSpec((NSC, RB, 16), lambda i: (0, i, 0)),
        ],
        out_specs=[
            pl.BlockSpec((2, RB, 32), lambda i: (0, i, 0)),
            pl.BlockSpec((2, RB, 32), lambda i: (0, i, 0)),
        ],
        out_shape=[
            jax.ShapeDtypeStruct((2, NP, 32), jnp.float32),
            jax.ShapeDtypeStruct((2, NP, 32), jnp.float32),
        ],
    )(xw, degp)


def _comb_body(gp_ref, u_ref, degp_ref, o_ref):
    dinv = _dinv_of(degp_ref[...])
    d2 = (dinv * dinv)[None]
    o_ref[...] = u_ref[...] - 2.0 * d2 * (gp_ref[0] + gp_ref[1])


def _comb(gp, u, degp):
    G, _, CD = u.shape
    return pl.pallas_call(
        _comb_body,
        grid=(pl.cdiv(NP, RB),),
        in_specs=[
            pl.BlockSpec((NSC, G, RB, CD), lambda i: (0, 0, i, 0)),
            pl.BlockSpec((G, RB, CD), lambda i: (0, i, 0)),
            pl.BlockSpec((NSC, RB, 16), lambda i: (0, i, 0)),
        ],
        out_specs=pl.BlockSpec((G, RB, CD), lambda i: (0, i, 0)),
        out_shape=jax.ShapeDtypeStruct((G, NP, CD), jnp.float32),
    )(gp, u, degp)


def _layer2_body(xw_ref, g1p_ref, degp_ref, b1_ref, wc2_ref, b2_ref,
                 b2v_ref, v1d_ref, y2b_ref, d2b2_ref):
    dinv = _dinv_of(degp_ref[...])
    a = xw_ref[:, 0:HID]
    g1 = g1p_ref[0] + g1p_ref[1]
    g1cat = jnp.concatenate([g1[0], g1[1]], axis=1)
    h = jnp.maximum(a + b1_ref[...] - dinv * g1cat, 0.0)
    hw = jnp.dot(h, wc2_ref[...], preferred_element_type=jnp.float32)
    b2v_ref[...] = hw[:, 0:16] + b2_ref[...]
    v1d_ref[...] = dinv * hw[:, 16:32]
    y2b_ref[...] = dinv * hw[:, 32:48]
    d2b2_ref[...] = jnp.broadcast_to(2.0 * dinv * dinv, d2b2_ref.shape)


def _layer2(xw, g1p, degp, b1r, wc2, b2r):
    return pl.pallas_call(
        _layer2_body,
        grid=(pl.cdiv(NP, RB),),
        in_specs=[
            pl.BlockSpec((RB, 3 * HID), lambda i: (i, 0)),
            pl.BlockSpec((NSC, 2, RB, 32), lambda i: (0, 0, i, 0)),
            pl.BlockSpec((NSC, RB, 16), lambda i: (0, i, 0)),
            pl.BlockSpec((1, HID), lambda i: (0, 0)),
            pl.BlockSpec((HID, 48), lambda i: (0, 0)),
            pl.BlockSpec((1, 16), lambda i: (0, 0)),
        ],
        out_specs=[
            pl.BlockSpec((RB, 16), lambda i: (i, 0)),
            pl.BlockSpec((RB, 16), lambda i: (i, 0)),
            pl.BlockSpec((RB, 16), lambda i: (i, 0)),
            pl.BlockSpec((RB, 16), lambda i: (i, 0)),
        ],
        out_shape=[
            jax.ShapeDtypeStruct((NP, 16), jnp.float32),
            jax.ShapeDtypeStruct((NP, 16), jnp.float32),
            jax.ShapeDtypeStruct((NP, 16), jnp.float32),
            jax.ShapeDtypeStruct((NP, 16), jnp.float32),
        ],
    )(xw, g1p, degp, b1r, wc2, b2r)


def _final_body(b2v_ref, g1bp_ref, degp_ref, o_ref):
    dinv = _dinv_of(degp_ref[...])
    o_ref[...] = b2v_ref[...] - dinv * (g1bp_ref[0] + g1bp_ref[1])


def _final(b2v, g1bp, degp):
    return pl.pallas_call(
        _final_body,
        grid=(pl.cdiv(NP, RB),),
        in_specs=[
            pl.BlockSpec((RB, 16), lambda i: (i, 0)),
            pl.BlockSpec((NSC, RB, 16), lambda i: (0, i, 0)),
            pl.BlockSpec((NSC, RB, 16), lambda i: (0, i, 0)),
        ],
        out_specs=pl.BlockSpec((RB, 16), lambda i: (i, 0)),
        out_shape=jax.ShapeDtypeStruct((NP, 16), jnp.float32),
    )(b2v, g1bp, degp)


def kernel(x, edge_index, W1, b1, W2, b2):
    row, col = edge_index[0], edge_index[1]
    pad = EPAD - E
    trash = jnp.full((pad,), N, dtype=jnp.int32)
    rowp = jnp.concatenate([row, trash]).reshape(NSC * NT, CPT, CH)
    colp = jnp.concatenate([col, trash]).reshape(NSC * NT, CPT, CH)

    wc1 = jnp.concatenate([W1[0] - W1[2], W1[1], W1[2]], axis=1)
    wc2 = jnp.zeros((HID, 48), jnp.float32)
    wc2 = wc2.at[:, 0:NCLS].set(W2[0] - W2[2])
    wc2 = wc2.at[:, 16:16 + NCLS].set(W2[1])
    wc2 = wc2.at[:, 32:32 + NCLS].set(W2[2])
    b1r = b1.reshape(1, HID)
    b2r = jnp.zeros((1, 16), jnp.float32).at[0, 0:NCLS].set(b2)

    onesc = jnp.ones((CH, 16), jnp.float32)
    zdeg = jnp.zeros((NP, 16), jnp.float32)
    zeros64 = jnp.zeros((2, NP, 32), jnp.float32)

    degp = _deg_pass()(rowp, onesc, zdeg)
    xw = _mm1(x, wc1)

    y2, u1d = _prep(xw, degp)
    g2p = _s_pass(HID)(y2, rowp, colp, zeros64)
    inner = _comb(g2p, u1d, degp)
    g1p = _s_pass(HID)(inner, rowp, colp, zeros64)

    b2v, v1d, y2b, d2b2 = _layer2(xw, g1p, degp, b1r, wc2, b2r)
    g1bp = _l2_pass()(y2b, v1d, d2b2, rowp, colp, zdeg)

    out16 = _final(b2v, g1bp, degp)
    return (out16[:N, :NCLS], edge_index)
